# baseline (speedup 1.0000x reference)
import functools

import jax
import jax.numpy as jnp
from jax.experimental import pallas as pl
from jax.experimental.pallas import tpu as pltpu

_K = 8
_BN = 8


def _wsum_kernel(*refs):
    x_refs = refs[:-2]
    w_ref = refs[-2]
    o_ref = refs[-1]
    ck = x_refs[0].shape[1]
    acc = None
    for k, x_ref in enumerate(x_refs):
        w = w_ref[k * ck:(k + 1) * ck, :]
        part = jnp.sum(x_ref[...] * w[None, :, :], axis=1)
        acc = part if acc is None else acc + part
    o_ref[...] = acc


def kernel(x_nchw, weight):
    N, C_in, H, W = x_nchw.shape
    C_out = weight.shape[0]
    HW = H * W
    w_col = weight.reshape(C_out * C_in, 1).astype(jnp.float32)

    k = _K
    while C_in % k or (C_in // k) % 8:
        k //= 2
    bn = _BN
    while N % bn:
        bn //= 2
    ck = C_in // k

    x = x_nchw.reshape(N, C_in, HW)
    in_specs = [
        pl.BlockSpec((bn, ck, HW),
                     functools.partial(lambda kk, i: (i, kk, 0), j))
        for j in range(k)
    ]
    in_specs.append(pl.BlockSpec((C_in, 1), lambda i: (0, 0)))

    in_bytes = bn * C_in * HW * x.dtype.itemsize
    vmem = int(min(2 * in_bytes + (2 * bn * HW + C_in) * 4 + (1 << 20),
                   100 << 20))

    out = pl.pallas_call(
        _wsum_kernel,
        out_shape=jax.ShapeDtypeStruct((N, HW), x_nchw.dtype),
        grid=(N // bn,),
        in_specs=in_specs,
        out_specs=pl.BlockSpec((bn, HW), lambda i: (i, 0)),
        compiler_params=pltpu.CompilerParams(
            dimension_semantics=("arbitrary",),
            vmem_limit_bytes=vmem,
        ),
    )(*([x] * k), w_col)
    return out.reshape(N, C_out, H, W)


# speedup vs baseline: 1.0317x; 1.0317x over previous
import functools

import jax
import jax.numpy as jnp
from jax.experimental import pallas as pl
from jax.experimental.pallas import tpu as pltpu

_K = 4
_BN = 4


def _wsum_kernel(*refs):
    n_streams = len(refs) // 2
    x_refs = refs[:n_streams]
    w_ref = refs[n_streams]
    o_refs = refs[n_streams + 1:]
    w = w_ref[...]
    for x_ref, o_ref in zip(x_refs, o_refs):
        o_ref[...] = jnp.sum(x_ref[...] * w[None, :, :], axis=1)


def kernel(x_nchw, weight):
    N, C_in, H, W = x_nchw.shape
    C_out = weight.shape[0]
    HW = H * W
    w_col = weight.reshape(C_out * C_in, 1).astype(jnp.float32)

    k, bn = _K, _BN
    while N % (k * bn):
        bn //= 2
    g = N // (k * bn)
    ns = N // k

    x = x_nchw.reshape(N, C_in, HW)
    in_specs = [
        pl.BlockSpec((bn, C_in, HW),
                     functools.partial(lambda jj, i: (jj * g + i, 0, 0), j))
        for j in range(k)
    ]
    in_specs.append(pl.BlockSpec((C_in, 1), lambda i: (0, 0)))

    out_shapes = [jax.ShapeDtypeStruct((g, bn, HW), x_nchw.dtype)
                  for _ in range(k)]
    out_specs = [pl.BlockSpec((None, bn, HW), lambda i: (i, 0, 0))
                 for _ in range(k)]

    in_bytes = k * bn * C_in * HW * x.dtype.itemsize
    vmem = int(min(2 * in_bytes + (2 * k * bn * HW + C_in) * 4 + (1 << 20),
                   100 << 20))

    outs = pl.pallas_call(
        _wsum_kernel,
        out_shape=out_shapes,
        grid=(g,),
        in_specs=in_specs,
        out_specs=out_specs,
        compiler_params=pltpu.CompilerParams(
            dimension_semantics=("arbitrary",),
            vmem_limit_bytes=vmem,
        ),
    )(*([x] * k), w_col)
    out = jnp.concatenate([o.reshape(ns, HW) for o in outs], axis=0)
    return out.reshape(N, C_out, H, W)
